# Initial kernel scaffold; baseline (speedup 1.0000x reference)
import jax
import jax.numpy as jnp
from jax.experimental import pallas as pl


def kernel(user_nodes, item_nodes, v_feat, words_tensor, edge_index, id_embedding, word_table, conv_weight, lin_W, lin_b, weight_W, weight_2):
    raise NotImplementedError("write your pallas kernel here")


# speedup vs baseline: 11.6218x; 11.6218x over previous
import functools

import jax
import jax.numpy as jnp
from jax import lax
from jax.experimental import pallas as pl
from jax.experimental.pallas import tpu as pltpu
from jax.experimental.pallas import tpu_sc as plsc

_NUM_USER = 5000
_NUM_ITEM = 5000
_N = _NUM_USER + _NUM_ITEM
_D = 128
_E = 320000
_B = 1024
_NWORDS = 100000

_NC = 2
_NS = 16
_NW = _NC * _NS
_CH = 128

_NW_PAD = ((_NWORDS + _CH - 1) // _CH) * _CH
_N_EDGE_CH = _E // _CH
_N_WORD_CH = _NW_PAD // _CH
_EDGE_T = (_N_EDGE_CH + _NW - 1) // _NW
_WORD_T = (_N_WORD_CH + _NW - 1) // _NW

_RS = 640
_RS_LAST = _N - 15 * _RS
_ZROWS = 80


def _leaky(x):
    return jnp.where(x >= 0, x, 0.01 * x)


def _norm_body(x_ref, o_ref):
    x = x_ref[...]
    n = jnp.sqrt(jnp.sum(x * x, axis=1, keepdims=True))
    o_ref[...] = x / jnp.maximum(n, 1e-12)


def _tc_normalize(x):
    return pl.pallas_call(
        _norm_body,
        grid=(10,),
        in_specs=[pl.BlockSpec((_N // 10, _D), lambda i: (i, 0))],
        out_specs=pl.BlockSpec((_N // 10, _D), lambda i: (i, 0)),
        out_shape=jax.ShapeDtypeStruct((_N, _D), jnp.float32),
    )(x)


def _sc_agg_body(x_hbm, src_hbm, dst_hbm, wi_hbm, ww_hbm, wtab_hbm,
                 agg_out, t_out, deg_out, wcnt_out,
                 src_idx, dst_idx, rows_a, rows_b, hist,
                 acc_sh, sem_a, sem_b):
    c = lax.axis_index("c")
    s = lax.axis_index("s")
    wid = s * _NC + c
    ones16 = jnp.ones((16,), jnp.float32)
    zeros16 = jnp.zeros((16,), jnp.float32)

    def _zero_hist():
        def _zh(i, carry):
            hist[pl.ds(i * 16, 16)] = zeros16
            return carry
        lax.fori_loop(0, _N // 16, _zh, None)

    def _zero_rows_a():
        def _zr(i, carry):
            for k in range(_D // 16):
                rows_a[i, pl.ds(k * 16, 16)] = zeros16
            return carry
        lax.fori_loop(0, _CH, _zr, None)

    def _zero_acc():
        zsrc = rows_a.at[pl.ds(0, _ZROWS), :]

        @pl.when(s < 15)
        def _():
            for q in range(_RS // _ZROWS):
                pltpu.sync_copy(
                    zsrc, acc_sh.at[pl.ds(s * _RS + q * _ZROWS, _ZROWS), :])

        @pl.when(s == 15)
        def _():
            for q in range(_RS_LAST // _ZROWS):
                pltpu.sync_copy(
                    zsrc, acc_sh.at[pl.ds(15 * _RS + q * _ZROWS, _ZROWS), :])

    _zero_hist()
    _zero_rows_a()
    _zero_acc()
    plsc.subcore_barrier()

    def _word_chunk(t, _):
        j = wid + _NW * t

        @pl.when(j < _N_WORD_CH)
        def _():
            e0 = j * _CH
            pltpu.sync_copy(wi_hbm.at[pl.ds(e0, _CH)], src_idx)
            pltpu.sync_copy(ww_hbm.at[pl.ds(e0, _CH)], dst_idx)
            pltpu.async_copy(wtab_hbm.at[dst_idx], rows_a, sem_a).wait()
            pltpu.sync_copy(rows_a, acc_sh.at[src_idx], add=True)
            for k in range(_CH // 16):
                plsc.addupdate_scatter(
                    hist, [src_idx[pl.ds(k * 16, 16)]], ones16)
        return _
    lax.fori_loop(0, _WORD_T, _word_chunk, None)
    plsc.subcore_barrier()

    @pl.when(s < 7)
    def _():
        pltpu.sync_copy(acc_sh.at[pl.ds(s * _RS, _RS), :],
                        t_out.at[c, pl.ds(s * _RS, _RS), :])

    @pl.when(s == 7)
    def _():
        pltpu.sync_copy(acc_sh.at[pl.ds(7 * _RS, _NUM_ITEM - 7 * _RS), :],
                        t_out.at[c, pl.ds(7 * _RS, _NUM_ITEM - 7 * _RS), :])
    pltpu.sync_copy(hist.at[pl.ds(0, _NUM_ITEM)], wcnt_out.at[wid, 0])
    plsc.subcore_barrier()

    _zero_hist()
    _zero_rows_a()
    _zero_acc()
    plsc.subcore_barrier()

    def _edge_chunk(t, _):
        j = wid + _NW * t

        @pl.when(j < _N_EDGE_CH)
        def _():
            e0 = j * _CH
            pltpu.sync_copy(src_hbm.at[pl.ds(e0, _CH)], src_idx)
            pltpu.sync_copy(dst_hbm.at[pl.ds(e0, _CH)], dst_idx)
            cp_a = pltpu.async_copy(x_hbm.at[src_idx], rows_a, sem_a)
            cp_b = pltpu.async_copy(x_hbm.at[dst_idx], rows_b, sem_b)
            cp_a.wait()
            cp_b.wait()
            pltpu.sync_copy(rows_a, acc_sh.at[dst_idx], add=True)
            pltpu.sync_copy(rows_b, acc_sh.at[src_idx], add=True)
            for k in range(_CH // 16):
                plsc.addupdate_scatter(
                    hist, [src_idx[pl.ds(k * 16, 16)]], ones16)
                plsc.addupdate_scatter(
                    hist, [dst_idx[pl.ds(k * 16, 16)]], ones16)
        return _
    lax.fori_loop(0, _EDGE_T, _edge_chunk, None)
    plsc.subcore_barrier()

    @pl.when(s < 15)
    def _():
        pltpu.sync_copy(acc_sh.at[pl.ds(s * _RS, _RS), :],
                        agg_out.at[c, pl.ds(s * _RS, _RS), :])

    @pl.when(s == 15)
    def _():
        pltpu.sync_copy(acc_sh.at[pl.ds(15 * _RS, _RS_LAST), :],
                        agg_out.at[c, pl.ds(15 * _RS, _RS_LAST), :])
    pltpu.sync_copy(hist, deg_out.at[wid, 0])


_sc_agg = functools.partial(
    pl.kernel,
    out_type=[
        jax.ShapeDtypeStruct((_NC, _N, _D), jnp.float32),
        jax.ShapeDtypeStruct((_NC, _NUM_ITEM, _D), jnp.float32),
        jax.ShapeDtypeStruct((_NW, 1, _N), jnp.float32),
        jax.ShapeDtypeStruct((_NW, 1, _NUM_ITEM), jnp.float32),
    ],
    mesh=plsc.VectorSubcoreMesh(core_axis_name="c", subcore_axis_name="s"),
    scratch_types=[
        pltpu.VMEM((_CH,), jnp.int32),
        pltpu.VMEM((_CH,), jnp.int32),
        pltpu.VMEM((_CH, _D), jnp.float32),
        pltpu.VMEM((_CH, _D), jnp.float32),
        pltpu.VMEM((_N,), jnp.float32),
        pltpu.VMEM_SHARED((_N, _D), jnp.float32),
        pltpu.SemaphoreType.DMA,
        pltpu.SemaphoreType.DMA,
    ],
    compiler_params=pltpu.CompilerParams(needs_layout_passes=False),
)(_sc_agg_body)


def _dense_body(aggp_ref, degp_ref, tp_ref, wcp_ref, v_ref,
                cw_ref, ww_ref, w2_ref, wv_ref, wt_ref, b_ref, o_ref):
    agg = (aggp_ref[0] + aggp_ref[1])
    deg = jnp.sum(degp_ref[...], axis=1)
    agg = agg / jnp.maximum(deg, 1.0)[:, None]
    x1 = _leaky(jnp.dot(agg, cw_ref[...], preferred_element_type=jnp.float32))
    y = jnp.dot(x1, ww_ref[...], preferred_element_type=jnp.float32)
    pid = pl.program_id(0)

    @pl.when(pid >= 5)
    def _():
        t_s = tp_ref[0] + tp_ref[1]
        wc = jnp.sum(wcp_ref[...], axis=1)
        t_feat = t_s / jnp.maximum(wc, 1.0)[:, None]
        f = _leaky(jnp.dot(v_ref[...], wv_ref[...],
                           preferred_element_type=jnp.float32)
                   + jnp.dot(t_feat, wt_ref[...],
                             preferred_element_type=jnp.float32)
                   + b_ref[...])
        o_ref[...] = _leaky(
            y + jnp.dot(f, w2_ref[...], preferred_element_type=jnp.float32))

    @pl.when(pid < 5)
    def _():
        o_ref[...] = _leaky(y)


def _tc_dense(aggp, degp_t, tp, wcp_t, v_feat, cw, ww, w2, wv, wt, b2d):
    R = _N // 10
    full = lambda i: (0, 0)
    return pl.pallas_call(
        _dense_body,
        grid=(10,),
        in_specs=[
            pl.BlockSpec((_NC, R, _D), lambda i: (0, i, 0)),
            pl.BlockSpec((R, _NW), lambda i: (i, 0)),
            pl.BlockSpec((_NC, R, _D), lambda i: (0, jnp.maximum(i - 5, 0), 0)),
            pl.BlockSpec((R, _NW), lambda i: (jnp.maximum(i - 5, 0), 0)),
            pl.BlockSpec((R, _D), lambda i: (jnp.maximum(i - 5, 0), 0)),
            pl.BlockSpec((_D, _D), full),
            pl.BlockSpec((_D, _D), full),
            pl.BlockSpec((_D, _D), full),
            pl.BlockSpec((_D, _D), full),
            pl.BlockSpec((_D, _D), full),
            pl.BlockSpec((1, _D), full),
        ],
        out_specs=pl.BlockSpec((R, _D), lambda i: (i, 0)),
        out_shape=jax.ShapeDtypeStruct((_N, _D), jnp.float32),
    )(aggp, degp_t, tp, wcp_t, v_feat, cw, ww, w2, wv, wt, b2d)


def _sc_dot_body(x2_hbm, u_hbm, it_hbm, out_hbm,
                 uidx, iidx, urows, irows, sbuf, sem_a, sem_b):
    c = lax.axis_index("c")
    s = lax.axis_index("s")
    wid = s * _NC + c
    per = _B // _NW
    base = wid * per
    pltpu.sync_copy(u_hbm.at[pl.ds(base, per)], uidx)
    pltpu.sync_copy(it_hbm.at[pl.ds(base, per)], iidx)
    cp_a = pltpu.async_copy(x2_hbm.at[uidx], urows, sem_a)
    cp_b = pltpu.async_copy(x2_hbm.at[iidx], irows, sem_b)
    cp_a.wait()
    cp_b.wait()

    lanes = lax.iota(jnp.int32, 16)
    for g in range(per // 16):
        v = jnp.zeros((16,), jnp.float32)
        for jj in range(16):
            j = g * 16 + jj
            acc = urows[j, pl.ds(0, 16)] * irows[j, pl.ds(0, 16)]
            for k in range(1, _D // 16):
                acc = acc + (urows[j, pl.ds(k * 16, 16)]
                             * irows[j, pl.ds(k * 16, 16)])
            v = jnp.where(lanes == jj, jnp.sum(acc), v)
        sbuf[pl.ds(g * 16, 16)] = v
    pltpu.sync_copy(sbuf, out_hbm.at[pl.ds(base, per)])


_sc_dot = functools.partial(
    pl.kernel,
    out_type=jax.ShapeDtypeStruct((_B,), jnp.float32),
    mesh=plsc.VectorSubcoreMesh(core_axis_name="c", subcore_axis_name="s"),
    scratch_types=[
        pltpu.VMEM((_B // _NW,), jnp.int32),
        pltpu.VMEM((_B // _NW,), jnp.int32),
        pltpu.VMEM((_B // _NW, _D), jnp.float32),
        pltpu.VMEM((_B // _NW, _D), jnp.float32),
        pltpu.VMEM((_B // _NW,), jnp.float32),
        pltpu.SemaphoreType.DMA,
        pltpu.SemaphoreType.DMA,
    ],
    compiler_params=pltpu.CompilerParams(needs_layout_passes=False),
)(_sc_dot_body)


def kernel(user_nodes, item_nodes, v_feat, words_tensor, edge_index,
           id_embedding, word_table, conv_weight, lin_W, lin_b,
           weight_W, weight_2):
    src = edge_index[:, 0].astype(jnp.int32)
    dst = edge_index[:, 1].astype(jnp.int32)
    npad = _NW_PAD - _NWORDS
    wi = jnp.concatenate([words_tensor[0].astype(jnp.int32),
                          jnp.full((npad,), _N - 1, jnp.int32)])
    ww = jnp.concatenate([words_tensor[1].astype(jnp.int32),
                          jnp.zeros((npad,), jnp.int32)])

    x_norm = _tc_normalize(id_embedding)
    aggp, tp, degp, wcp = _sc_agg(x_norm, src, dst, wi, ww, word_table)
    x2 = _tc_dense(aggp, degp.reshape(_NW, _N).T, tp,
                   wcp.reshape(_NW, _NUM_ITEM).T, v_feat,
                   conv_weight, weight_W, weight_2,
                   lin_W[:, :_D].T, lin_W[:, _D:].T,
                   lin_b.reshape(1, _D))
    return _sc_dot(x2, user_nodes.astype(jnp.int32),
                   item_nodes.astype(jnp.int32))


# speedup vs baseline: 13.8902x; 1.1952x over previous
import functools

import jax
import jax.numpy as jnp
from jax import lax
from jax.experimental import pallas as pl
from jax.experimental.pallas import tpu as pltpu
from jax.experimental.pallas import tpu_sc as plsc

_NUM_USER = 5000
_NUM_ITEM = 5000
_N = _NUM_USER + _NUM_ITEM
_D = 128
_E = 320000
_B = 1024
_NWORDS = 100000

_NC = 2
_NS = 16
_NW = _NC * _NS
_CH = 64

_NW_PAD = ((_NWORDS + _CH - 1) // _CH) * _CH
_N_EDGE_CH = _E // _CH
_N_WORD_CH = _NW_PAD // _CH
_EDGE_T = (_N_EDGE_CH + _NW - 1) // _NW
_WORD_T = (_N_WORD_CH + _NW - 1) // _NW
_EDGE_T2 = (_EDGE_T + 1) // 2
_WORD_T2 = (_WORD_T + 1) // 2

_RS = 640
_RS_LAST = _N - 15 * _RS
_ZROWS = 80


def _leaky(x):
    return jnp.where(x >= 0, x, 0.01 * x)


def _norm_body(x_ref, o_ref):
    x = x_ref[...]
    n = jnp.sqrt(jnp.sum(x * x, axis=1, keepdims=True))
    o_ref[...] = x / jnp.maximum(n, 1e-12)


def _tc_normalize(x):
    return pl.pallas_call(
        _norm_body,
        grid=(10,),
        in_specs=[pl.BlockSpec((_N // 10, _D), lambda i: (i, 0))],
        out_specs=pl.BlockSpec((_N // 10, _D), lambda i: (i, 0)),
        out_shape=jax.ShapeDtypeStruct((_N, _D), jnp.float32),
    )(x)


def _sc_agg_body(x_hbm, src_hbm, dst_hbm, wi_hbm, ww_hbm, wtab_hbm,
                 agg_out, t_out, deg_out, wcnt_out,
                 si0, di0, si1, di1, ra0, rb0, ra1, rb1, hist, acc_sh,
                 ga0, gb0, ga1, gb1, sa0, sb0, sa1, sb1):
    c = lax.axis_index("c")
    s = lax.axis_index("s")
    wid = s * _NC + c
    ones16 = jnp.ones((16,), jnp.float32)
    zeros16 = jnp.zeros((16,), jnp.float32)

    def _zero_hist():
        def _zh(i, carry):
            hist[pl.ds(i * 16, 16)] = zeros16
            return carry
        lax.fori_loop(0, _N // 16, _zh, None)

    def _zero_ra0():
        def _zr(i, carry):
            for k in range(_D // 16):
                ra0[i, pl.ds(k * 16, 16)] = zeros16
            return carry
        lax.fori_loop(0, _CH, _zr, None)

    def _zero_acc():
        @pl.when(s < 15)
        def _():
            for q in range(_RS // _CH):
                pltpu.sync_copy(
                    ra0, acc_sh.at[pl.ds(s * _RS + q * _CH, _CH), :])

        @pl.when(s == 15)
        def _():
            for q in range(_RS_LAST // _CH):
                pltpu.sync_copy(
                    ra0, acc_sh.at[pl.ds(15 * _RS + q * _CH, _CH), :])
            rem = _RS_LAST - (_RS_LAST // _CH) * _CH
            if rem:
                pltpu.sync_copy(
                    ra0.at[pl.ds(0, rem), :],
                    acc_sh.at[pl.ds(_N - rem, rem), :])

    _zero_hist()
    _zero_ra0()
    _zero_acc()
    plsc.subcore_barrier()

    def w_start(t, si, di, ra, ga):
        j = wid + _NW * t

        @pl.when((j >= 0) & (j < _N_WORD_CH))
        def _():
            e0 = j * _CH
            pltpu.sync_copy(wi_hbm.at[pl.ds(e0, _CH)], si)
            pltpu.sync_copy(ww_hbm.at[pl.ds(e0, _CH)], di)
            pltpu.async_copy(wtab_hbm.at[di], ra, ga)

    def w_finish(t, si, di, ra, ga, sa):
        j = wid + _NW * t

        @pl.when((j >= 0) & (j < _N_WORD_CH))
        def _():
            pltpu.make_async_copy(wtab_hbm.at[di], ra, ga).wait()
            pltpu.async_copy(ra, acc_sh.at[si], sa, add=True)
            for k in range(_CH // 16):
                plsc.addupdate_scatter(hist, [si[pl.ds(k * 16, 16)]], ones16)

    def w_drain(t, si, ra, sa):
        j = wid + _NW * t

        @pl.when((j >= 0) & (j < _N_WORD_CH))
        def _():
            pltpu.make_async_copy(ra, acc_sh.at[si], sa).wait()

    w_start(0, si0, di0, ra0, ga0)

    def _word_pair(t2, carry):
        te = 2 * t2
        to = te + 1
        w_drain(to - 2, si1, ra1, sa1)
        w_start(to, si1, di1, ra1, ga1)
        w_finish(te, si0, di0, ra0, ga0, sa0)
        w_finish(to, si1, di1, ra1, ga1, sa1)
        w_drain(te, si0, ra0, sa0)
        w_start(te + 2, si0, di0, ra0, ga0)
        return carry
    lax.fori_loop(0, _WORD_T2, _word_pair, None)
    w_drain(2 * _WORD_T2 - 1, si1, ra1, sa1)
    plsc.subcore_barrier()

    @pl.when(s < 7)
    def _():
        pltpu.sync_copy(acc_sh.at[pl.ds(s * _RS, _RS), :],
                        t_out.at[c, pl.ds(s * _RS, _RS), :])

    @pl.when(s == 7)
    def _():
        pltpu.sync_copy(acc_sh.at[pl.ds(7 * _RS, _NUM_ITEM - 7 * _RS), :],
                        t_out.at[c, pl.ds(7 * _RS, _NUM_ITEM - 7 * _RS), :])
    pltpu.sync_copy(hist.at[pl.ds(0, _NUM_ITEM)], wcnt_out.at[wid, 0])
    plsc.subcore_barrier()

    _zero_hist()
    _zero_ra0()
    _zero_acc()
    plsc.subcore_barrier()

    def e_start(t, si, di, ra, rb, ga, gb):
        j = wid + _NW * t

        @pl.when((j >= 0) & (j < _N_EDGE_CH))
        def _():
            e0 = j * _CH
            pltpu.sync_copy(src_hbm.at[pl.ds(e0, _CH)], si)
            pltpu.sync_copy(dst_hbm.at[pl.ds(e0, _CH)], di)
            pltpu.async_copy(x_hbm.at[si], ra, ga)
            pltpu.async_copy(x_hbm.at[di], rb, gb)

    def e_finish(t, si, di, ra, rb, ga, gb, sa, sb):
        j = wid + _NW * t

        @pl.when((j >= 0) & (j < _N_EDGE_CH))
        def _():
            pltpu.make_async_copy(x_hbm.at[si], ra, ga).wait()
            pltpu.make_async_copy(x_hbm.at[di], rb, gb).wait()
            pltpu.async_copy(ra, acc_sh.at[di], sa, add=True)
            pltpu.async_copy(rb, acc_sh.at[si], sb, add=True)
            for k in range(_CH // 16):
                plsc.addupdate_scatter(hist, [si[pl.ds(k * 16, 16)]], ones16)
                plsc.addupdate_scatter(hist, [di[pl.ds(k * 16, 16)]], ones16)

    def e_drain(t, si, di, ra, rb, sa, sb):
        j = wid + _NW * t

        @pl.when((j >= 0) & (j < _N_EDGE_CH))
        def _():
            pltpu.make_async_copy(ra, acc_sh.at[di], sa).wait()
            pltpu.make_async_copy(rb, acc_sh.at[si], sb).wait()

    e_start(0, si0, di0, ra0, rb0, ga0, gb0)

    def _edge_pair(t2, carry):
        te = 2 * t2
        to = te + 1
        e_drain(to - 2, si1, di1, ra1, rb1, sa1, sb1)
        e_start(to, si1, di1, ra1, rb1, ga1, gb1)
        e_finish(te, si0, di0, ra0, rb0, ga0, gb0, sa0, sb0)
        e_finish(to, si1, di1, ra1, rb1, ga1, gb1, sa1, sb1)
        e_drain(te, si0, di0, ra0, rb0, sa0, sb0)
        e_start(te + 2, si0, di0, ra0, rb0, ga0, gb0)
        return carry
    lax.fori_loop(0, _EDGE_T2, _edge_pair, None)
    e_drain(2 * _EDGE_T2 - 1, si1, di1, ra1, rb1, sa1, sb1)
    plsc.subcore_barrier()

    @pl.when(s < 15)
    def _():
        pltpu.sync_copy(acc_sh.at[pl.ds(s * _RS, _RS), :],
                        agg_out.at[c, pl.ds(s * _RS, _RS), :])

    @pl.when(s == 15)
    def _():
        pltpu.sync_copy(acc_sh.at[pl.ds(15 * _RS, _RS_LAST), :],
                        agg_out.at[c, pl.ds(15 * _RS, _RS_LAST), :])
    pltpu.sync_copy(hist, deg_out.at[wid, 0])


_sc_agg = functools.partial(
    pl.kernel,
    out_type=[
        jax.ShapeDtypeStruct((_NC, _N, _D), jnp.float32),
        jax.ShapeDtypeStruct((_NC, _NUM_ITEM, _D), jnp.float32),
        jax.ShapeDtypeStruct((_NW, 1, _N), jnp.float32),
        jax.ShapeDtypeStruct((_NW, 1, _NUM_ITEM), jnp.float32),
    ],
    mesh=plsc.VectorSubcoreMesh(core_axis_name="c", subcore_axis_name="s"),
    scratch_types=[
        pltpu.VMEM((_CH,), jnp.int32),
        pltpu.VMEM((_CH,), jnp.int32),
        pltpu.VMEM((_CH,), jnp.int32),
        pltpu.VMEM((_CH,), jnp.int32),
        pltpu.VMEM((_CH, _D), jnp.float32),
        pltpu.VMEM((_CH, _D), jnp.float32),
        pltpu.VMEM((_CH, _D), jnp.float32),
        pltpu.VMEM((_CH, _D), jnp.float32),
        pltpu.VMEM((_N,), jnp.float32),
        pltpu.VMEM_SHARED((_N, _D), jnp.float32),
        pltpu.SemaphoreType.DMA,
        pltpu.SemaphoreType.DMA,
        pltpu.SemaphoreType.DMA,
        pltpu.SemaphoreType.DMA,
        pltpu.SemaphoreType.DMA,
        pltpu.SemaphoreType.DMA,
        pltpu.SemaphoreType.DMA,
        pltpu.SemaphoreType.DMA,
    ],
    compiler_params=pltpu.CompilerParams(needs_layout_passes=False),
)(_sc_agg_body)


def _dense_body(aggp_ref, degp_ref, tp_ref, wcp_ref, v_ref,
                cw_ref, ww_ref, w2_ref, wv_ref, wt_ref, b_ref, o_ref):
    agg = (aggp_ref[0] + aggp_ref[1])
    deg = jnp.sum(degp_ref[...], axis=1)
    agg = agg / jnp.maximum(deg, 1.0)[:, None]
    x1 = _leaky(jnp.dot(agg, cw_ref[...], preferred_element_type=jnp.float32))
    y = jnp.dot(x1, ww_ref[...], preferred_element_type=jnp.float32)
    pid = pl.program_id(0)

    @pl.when(pid >= 5)
    def _():
        t_s = tp_ref[0] + tp_ref[1]
        wc = jnp.sum(wcp_ref[...], axis=1)
        t_feat = t_s / jnp.maximum(wc, 1.0)[:, None]
        f = _leaky(jnp.dot(v_ref[...], wv_ref[...],
                           preferred_element_type=jnp.float32)
                   + jnp.dot(t_feat, wt_ref[...],
                             preferred_element_type=jnp.float32)
                   + b_ref[...])
        o_ref[...] = _leaky(
            y + jnp.dot(f, w2_ref[...], preferred_element_type=jnp.float32))

    @pl.when(pid < 5)
    def _():
        o_ref[...] = _leaky(y)


def _tc_dense(aggp, degp_t, tp, wcp_t, v_feat, cw, ww, w2, wv, wt, b2d):
    R = _N // 10
    full = lambda i: (0, 0)
    return pl.pallas_call(
        _dense_body,
        grid=(10,),
        in_specs=[
            pl.BlockSpec((_NC, R, _D), lambda i: (0, i, 0)),
            pl.BlockSpec((R, _NW), lambda i: (i, 0)),
            pl.BlockSpec((_NC, R, _D), lambda i: (0, jnp.maximum(i - 5, 0), 0)),
            pl.BlockSpec((R, _NW), lambda i: (jnp.maximum(i - 5, 0), 0)),
            pl.BlockSpec((R, _D), lambda i: (jnp.maximum(i - 5, 0), 0)),
            pl.BlockSpec((_D, _D), full),
            pl.BlockSpec((_D, _D), full),
            pl.BlockSpec((_D, _D), full),
            pl.BlockSpec((_D, _D), full),
            pl.BlockSpec((_D, _D), full),
            pl.BlockSpec((1, _D), full),
        ],
        out_specs=pl.BlockSpec((R, _D), lambda i: (i, 0)),
        out_shape=jax.ShapeDtypeStruct((_N, _D), jnp.float32),
    )(aggp, degp_t, tp, wcp_t, v_feat, cw, ww, w2, wv, wt, b2d)


def _sc_dot_body(x2_hbm, u_hbm, it_hbm, out_hbm,
                 uidx, iidx, urows, irows, sbuf, sem_a, sem_b):
    c = lax.axis_index("c")
    s = lax.axis_index("s")
    wid = s * _NC + c
    per = _B // _NW
    base = wid * per
    pltpu.sync_copy(u_hbm.at[pl.ds(base, per)], uidx)
    pltpu.sync_copy(it_hbm.at[pl.ds(base, per)], iidx)
    cp_a = pltpu.async_copy(x2_hbm.at[uidx], urows, sem_a)
    cp_b = pltpu.async_copy(x2_hbm.at[iidx], irows, sem_b)
    cp_a.wait()
    cp_b.wait()

    lanes = lax.iota(jnp.int32, 16)
    for g in range(per // 16):
        v = jnp.zeros((16,), jnp.float32)
        for jj in range(16):
            j = g * 16 + jj
            acc = urows[j, pl.ds(0, 16)] * irows[j, pl.ds(0, 16)]
            for k in range(1, _D // 16):
                acc = acc + (urows[j, pl.ds(k * 16, 16)]
                             * irows[j, pl.ds(k * 16, 16)])
            v = jnp.where(lanes == jj, jnp.sum(acc), v)
        sbuf[pl.ds(g * 16, 16)] = v
    pltpu.sync_copy(sbuf, out_hbm.at[pl.ds(base, per)])


_sc_dot = functools.partial(
    pl.kernel,
    out_type=jax.ShapeDtypeStruct((_B,), jnp.float32),
    mesh=plsc.VectorSubcoreMesh(core_axis_name="c", subcore_axis_name="s"),
    scratch_types=[
        pltpu.VMEM((_B // _NW,), jnp.int32),
        pltpu.VMEM((_B // _NW,), jnp.int32),
        pltpu.VMEM((_B // _NW, _D), jnp.float32),
        pltpu.VMEM((_B // _NW, _D), jnp.float32),
        pltpu.VMEM((_B // _NW,), jnp.float32),
        pltpu.SemaphoreType.DMA,
        pltpu.SemaphoreType.DMA,
    ],
    compiler_params=pltpu.CompilerParams(needs_layout_passes=False),
)(_sc_dot_body)


def kernel(user_nodes, item_nodes, v_feat, words_tensor, edge_index,
           id_embedding, word_table, conv_weight, lin_W, lin_b,
           weight_W, weight_2):
    src = edge_index[:, 0].astype(jnp.int32)
    dst = edge_index[:, 1].astype(jnp.int32)
    npad = _NW_PAD - _NWORDS
    wi = jnp.concatenate([words_tensor[0].astype(jnp.int32),
                          jnp.full((npad,), _N - 1, jnp.int32)])
    ww = jnp.concatenate([words_tensor[1].astype(jnp.int32),
                          jnp.zeros((npad,), jnp.int32)])

    x_norm = _tc_normalize(id_embedding)
    aggp, tp, degp, wcp = _sc_agg(x_norm, src, dst, wi, ww, word_table)
    x2 = _tc_dense(aggp, degp.reshape(_NW, _N).T, tp,
                   wcp.reshape(_NW, _NUM_ITEM).T, v_feat,
                   conv_weight, weight_W, weight_2,
                   lin_W[:, :_D].T, lin_W[:, _D:].T,
                   lin_b.reshape(1, _D))
    return _sc_dot(x2, user_nodes.astype(jnp.int32),
                   item_nodes.astype(jnp.int32))
